# baseline (device time: 162553 ns/iter reference)
import jax
import jax.numpy as jnp
from jax import lax
from jax.experimental import pallas as pl
from jax.experimental.pallas import tpu as pltpu

N_DEV = 8
E_LOC = 4
CAP = 64
BLK = E_LOC * CAP


def _moe_fused(x, slot_col, slot_row, p, w_shard, shared_W):
    n_tok, d = x.shape
    e_loc, _, h_dim = w_shard.shape

    def body(x_ref, sc_ref, sr_ref, p_ref, w_ref, sw_ref, out_ref,
             bins_ref, r_ref, y_ref, back_ref,
             send_sems, recv_sems, back_send, back_recv):
        me = lax.axis_index("i")

        barrier_sem = pltpu.get_barrier_semaphore()
        for delta in range(1, N_DEV):
            pl.semaphore_signal(
                barrier_sem, inc=1,
                device_id=(lax.rem(me + delta, N_DEV),),
                device_id_type=pl.DeviceIdType.MESH,
            )
        pl.semaphore_wait(barrier_sem, N_DEV - 1)

        x_v = x_ref[...]
        slot_r = sr_ref[...]
        for t in range(N_DEV):
            iota_t = lax.broadcasted_iota(jnp.int32, (BLK, n_tok), 0) + t * BLK
            d_t = (iota_t == slot_r).astype(jnp.float32)
            bins_ref[t] = jnp.dot(
                d_t, x_v, preferred_element_type=jnp.float32
            ).reshape(E_LOC, CAP, d)

        r_ref[pl.ds(me, 1)] = bins_ref[pl.ds(me, 1)]
        sends = []
        for delta in range(1, N_DEV):
            t = lax.rem(me + delta, N_DEV)
            rdma = pltpu.make_async_remote_copy(
                src_ref=bins_ref.at[t],
                dst_ref=r_ref.at[me],
                send_sem=send_sems.at[t],
                recv_sem=recv_sems.at[me],
                device_id=(t,),
                device_id_type=pl.DeviceIdType.MESH,
            )
            rdma.start()
            sends.append(rdma)

        shared = jnp.dot(x_v, sw_ref[...], preferred_element_type=jnp.float32)

        for delta in range(1, N_DEV):
            s = lax.rem(me + N_DEV - delta, N_DEV)
            recv = pltpu.make_async_remote_copy(
                src_ref=bins_ref.at[s],
                dst_ref=r_ref.at[s],
                send_sem=send_sems.at[s],
                recv_sem=recv_sems.at[s],
                device_id=(s,),
                device_id_type=pl.DeviceIdType.MESH,
            )
            recv.wait_recv()

        for e in range(e_loc):
            x_e = r_ref[:, e].reshape(N_DEV * CAP, d)
            y_e = jnp.dot(x_e, w_ref[e], preferred_element_type=jnp.float32)
            y_ref[:, e] = y_e.reshape(N_DEV, CAP, h_dim)

        back_ref[pl.ds(me, 1)] = y_ref[pl.ds(me, 1)]
        for delta in range(1, N_DEV):
            t = lax.rem(me + delta, N_DEV)
            rdma = pltpu.make_async_remote_copy(
                src_ref=y_ref.at[t],
                dst_ref=back_ref.at[me],
                send_sem=back_send.at[t],
                recv_sem=back_recv.at[me],
                device_id=(t,),
                device_id_type=pl.DeviceIdType.MESH,
            )
            rdma.start()
            sends.append(rdma)
        for delta in range(1, N_DEV):
            s = lax.rem(me + N_DEV - delta, N_DEV)
            recv = pltpu.make_async_remote_copy(
                src_ref=y_ref.at[s],
                dst_ref=back_ref.at[s],
                send_sem=back_send.at[s],
                recv_sem=back_recv.at[s],
                device_id=(s,),
                device_id_type=pl.DeviceIdType.MESH,
            )
            recv.wait_recv()

        slot_c = sc_ref[...]
        acc = shared
        for t in range(N_DEV):
            iota_t = lax.broadcasted_iota(jnp.int32, (n_tok, BLK), 1) + t * BLK
            d_t = (iota_t == slot_c).astype(jnp.float32)
            yb_t = back_ref[t].reshape(BLK, h_dim)
            acc = acc + p_ref[...] * jnp.dot(
                d_t, yb_t, preferred_element_type=jnp.float32
            )
        out_ref[...] = acc

        for rdma in sends:
            rdma.wait_send()

    return pl.pallas_call(
        body,
        out_shape=jax.ShapeDtypeStruct((n_tok, h_dim), jnp.float32),
        in_specs=[pl.BlockSpec(memory_space=pltpu.VMEM)] * 6,
        out_specs=pl.BlockSpec(memory_space=pltpu.VMEM),
        scratch_shapes=[
            pltpu.VMEM((N_DEV, E_LOC, CAP, d), jnp.float32),
            pltpu.VMEM((N_DEV, E_LOC, CAP, d), jnp.float32),
            pltpu.VMEM((N_DEV, E_LOC, CAP, h_dim), jnp.float32),
            pltpu.VMEM((N_DEV, E_LOC, CAP, h_dim), jnp.float32),
            pltpu.SemaphoreType.DMA((N_DEV,)),
            pltpu.SemaphoreType.DMA((N_DEV,)),
            pltpu.SemaphoreType.DMA((N_DEV,)),
            pltpu.SemaphoreType.DMA((N_DEV,)),
        ],
        compiler_params=pltpu.CompilerParams(
            collective_id=0,
            vmem_limit_bytes=60 * 1024 * 1024,
        ),
    )(x, slot_col, slot_row, p, w_shard, shared_W)


def kernel(x, router_W, route_idx, expert_W, shared_W):
    n_tok, _ = x.shape
    n_exp = router_W.shape[1]

    e = route_idx[:, 0].astype(jnp.int32)
    one_hot_e = (e[:, None] == jnp.arange(n_exp, dtype=jnp.int32)[None, :])
    pos = (
        jnp.take_along_axis(
            jnp.cumsum(one_hot_e.astype(jnp.int32), axis=0), e[:, None], axis=1
        )[:, 0]
        - 1
    )
    slot = e * CAP + jnp.minimum(pos, CAP - 1)

    probs = jax.nn.softmax(x @ router_W, axis=-1)
    p = jnp.take_along_axis(probs, route_idx, axis=1)

    return _moe_fused(
        x, slot[:, None], slot[None, :], p, expert_W, shared_W
    )


# device time: 108217 ns/iter; 1.5021x vs baseline; 1.5021x over previous
import jax
import jax.numpy as jnp
from jax import lax
from jax.experimental import pallas as pl
from jax.experimental.pallas import tpu as pltpu

N_DEV = 8
E_LOC = 4
CAP = 64


def _moe_a2a(bins, w_shard):
    _, _, _, d = bins.shape
    e_loc, _, h_dim = w_shard.shape

    def body(bins_ref, w_ref, out_ref, r_ref, y_ref,
             send_sems, recv_sems, back_send, back_recv):
        me = lax.axis_index("i")

        barrier_sem = pltpu.get_barrier_semaphore()
        for delta in range(1, N_DEV):
            pl.semaphore_signal(
                barrier_sem, inc=1,
                device_id=(lax.rem(me + delta, N_DEV),),
                device_id_type=pl.DeviceIdType.MESH,
            )
        pl.semaphore_wait(barrier_sem, N_DEV - 1)

        r_ref[pl.ds(me, 1)] = bins_ref[pl.ds(me, 1)]
        sends = []
        for delta in range(1, N_DEV):
            t = lax.rem(me + delta, N_DEV)
            rdma = pltpu.make_async_remote_copy(
                src_ref=bins_ref.at[t],
                dst_ref=r_ref.at[me],
                send_sem=send_sems.at[t],
                recv_sem=recv_sems.at[me],
                device_id=(t,),
                device_id_type=pl.DeviceIdType.MESH,
            )
            rdma.start()
            sends.append(rdma)
        for delta in range(1, N_DEV):
            s = lax.rem(me + N_DEV - delta, N_DEV)
            recv = pltpu.make_async_remote_copy(
                src_ref=bins_ref.at[s],
                dst_ref=r_ref.at[s],
                send_sem=send_sems.at[s],
                recv_sem=recv_sems.at[s],
                device_id=(s,),
                device_id_type=pl.DeviceIdType.MESH,
            )
            recv.wait_recv()

        for e in range(e_loc):
            x_e = r_ref[:, e].reshape(N_DEV * CAP, d)
            y_e = jnp.dot(x_e, w_ref[e], preferred_element_type=jnp.float32)
            y_ref[:, e] = y_e.astype(jnp.bfloat16).reshape(N_DEV, CAP, h_dim)

        out_ref[pl.ds(me, 1)] = y_ref[pl.ds(me, 1)]
        for delta in range(1, N_DEV):
            t = lax.rem(me + delta, N_DEV)
            rdma = pltpu.make_async_remote_copy(
                src_ref=y_ref.at[t],
                dst_ref=out_ref.at[me],
                send_sem=back_send.at[t],
                recv_sem=back_recv.at[me],
                device_id=(t,),
                device_id_type=pl.DeviceIdType.MESH,
            )
            rdma.start()
            sends.append(rdma)
        for delta in range(1, N_DEV):
            s = lax.rem(me + N_DEV - delta, N_DEV)
            recv = pltpu.make_async_remote_copy(
                src_ref=y_ref.at[s],
                dst_ref=out_ref.at[s],
                send_sem=back_send.at[s],
                recv_sem=back_recv.at[s],
                device_id=(s,),
                device_id_type=pl.DeviceIdType.MESH,
            )
            recv.wait_recv()

        for rdma in sends:
            rdma.wait_send()

    return pl.pallas_call(
        body,
        out_shape=jax.ShapeDtypeStruct((N_DEV, e_loc, CAP, h_dim), jnp.bfloat16),
        in_specs=[
            pl.BlockSpec(memory_space=pltpu.VMEM),
            pl.BlockSpec(memory_space=pltpu.VMEM),
        ],
        out_specs=pl.BlockSpec(memory_space=pltpu.VMEM),
        scratch_shapes=[
            pltpu.VMEM((N_DEV, E_LOC, CAP, d), jnp.bfloat16),
            pltpu.VMEM((N_DEV, E_LOC, CAP, h_dim), jnp.bfloat16),
            pltpu.SemaphoreType.DMA((N_DEV,)),
            pltpu.SemaphoreType.DMA((N_DEV,)),
            pltpu.SemaphoreType.DMA((N_DEV,)),
            pltpu.SemaphoreType.DMA((N_DEV,)),
        ],
        compiler_params=pltpu.CompilerParams(
            collective_id=0,
            vmem_limit_bytes=60 * 1024 * 1024,
        ),
    )(bins, w_shard)


def kernel(x, router_W, route_idx, expert_W, shared_W):
    n_tok, d_model = x.shape
    n_exp = router_W.shape[1]
    h_dim = shared_W.shape[1]

    e = route_idx[:, 0].astype(jnp.int32)
    one_hot_e = (e[:, None] == jnp.arange(n_exp, dtype=jnp.int32)[None, :])
    pos = (
        jnp.take_along_axis(
            jnp.cumsum(one_hot_e.astype(jnp.int32), axis=0), e[:, None], axis=1
        )[:, 0]
        - 1
    )
    slot = e * CAP + jnp.minimum(pos, CAP - 1)
    disp = (
        slot[:, None] == jnp.arange(n_exp * CAP, dtype=jnp.int32)[None, :]
    ).astype(jnp.bfloat16)
    bins = (disp.T @ x.astype(jnp.bfloat16)).reshape(
        N_DEV, E_LOC, CAP, d_model
    )

    y_slots = _moe_a2a(bins, expert_W.astype(jnp.bfloat16))

    y = jnp.dot(
        disp, y_slots.reshape(n_exp * CAP, h_dim),
        preferred_element_type=jnp.float32,
    )

    probs = jax.nn.softmax(x @ router_W, axis=-1)
    p = jnp.take_along_axis(probs, route_idx, axis=1)
    return x @ shared_W + p * y


# device time: 79614 ns/iter; 2.0418x vs baseline; 1.3593x over previous
import jax
import jax.numpy as jnp
from jax import lax
from jax.experimental import pallas as pl
from jax.experimental.pallas import tpu as pltpu

N_DEV = 8
E_LOC = 4
CAP = 64
BLK = E_LOC * CAP


def _moe_fused(x, e_col, e_row, router_W, w_shard, shared_W):
    n_tok, d = x.shape
    e_loc, _, h_dim = w_shard.shape

    def body(x_ref, ec_ref, er_ref, rw_ref, w_ref, sw_ref, out_ref,
             bins_ref, r_ref, y_ref, back_ref,
             send_sems, recv_sems, back_send, back_recv):
        me = lax.axis_index("i")

        barrier_sem = pltpu.get_barrier_semaphore()
        for delta in range(1, N_DEV):
            pl.semaphore_signal(
                barrier_sem, inc=1,
                device_id=(lax.rem(me + delta, N_DEV),),
                device_id_type=pl.DeviceIdType.MESH,
            )
        pl.semaphore_wait(barrier_sem, N_DEV - 1)

        e_c = ec_ref[...]
        e_r = er_ref[...]

        eq = (e_c == e_r).astype(jnp.float32)
        ii = lax.broadcasted_iota(jnp.int32, (n_tok, n_tok), 0)
        jj = lax.broadcasted_iota(jnp.int32, (n_tok, n_tok), 1)
        lower = (jj <= ii).astype(jnp.float32)
        pos_c = jnp.sum(eq * lower, axis=1, keepdims=True).astype(jnp.int32) - 1
        pos_r = (
            jnp.sum(eq * (ii <= jj).astype(jnp.float32), axis=0, keepdims=True)
            .astype(jnp.int32)
            - 1
        )
        slot_c = e_c * CAP + jnp.minimum(pos_c, CAP - 1)
        slot_r = e_r * CAP + jnp.minimum(pos_r, CAP - 1)

        x_v = x_ref[...]
        for t in range(N_DEV):
            iota_t = lax.broadcasted_iota(jnp.int32, (BLK, n_tok), 0) + t * BLK
            d_t = (iota_t == slot_r).astype(jnp.float32)
            bins_ref[t] = (
                jnp.dot(d_t, x_v, preferred_element_type=jnp.float32)
                .astype(jnp.bfloat16)
                .reshape(E_LOC, CAP, d)
            )

        r_ref[pl.ds(me, 1)] = bins_ref[pl.ds(me, 1)]
        sends = []
        for delta in range(1, N_DEV):
            t = lax.rem(me + delta, N_DEV)
            rdma = pltpu.make_async_remote_copy(
                src_ref=bins_ref.at[t],
                dst_ref=r_ref.at[me],
                send_sem=send_sems.at[t],
                recv_sem=recv_sems.at[me],
                device_id=(t,),
                device_id_type=pl.DeviceIdType.MESH,
            )
            rdma.start()
            sends.append(rdma)

        shared = jnp.dot(x_v, sw_ref[...], preferred_element_type=jnp.float32)
        scores = jnp.dot(x_v, rw_ref[...], preferred_element_type=jnp.float32)
        m = jnp.max(scores, axis=1, keepdims=True)
        ex = jnp.exp(scores - m)
        onehot_e = (
            e_c == lax.broadcasted_iota(jnp.int32, (n_tok, scores.shape[1]), 1)
        ).astype(jnp.float32)
        p_col = jnp.sum(ex * onehot_e, axis=1, keepdims=True) / jnp.sum(
            ex, axis=1, keepdims=True
        )

        for delta in range(1, N_DEV):
            s = lax.rem(me + N_DEV - delta, N_DEV)
            recv = pltpu.make_async_remote_copy(
                src_ref=bins_ref.at[s],
                dst_ref=r_ref.at[s],
                send_sem=send_sems.at[s],
                recv_sem=recv_sems.at[s],
                device_id=(s,),
                device_id_type=pl.DeviceIdType.MESH,
            )
            recv.wait_recv()

        for e in range(e_loc):
            x_e = r_ref[:, e].reshape(N_DEV * CAP, d).astype(jnp.float32)
            y_e = jnp.dot(x_e, w_ref[e], preferred_element_type=jnp.float32)
            y_ref[:, e] = y_e.astype(jnp.bfloat16).reshape(N_DEV, CAP, h_dim)

        back_ref[pl.ds(me, 1)] = y_ref[pl.ds(me, 1)]
        for delta in range(1, N_DEV):
            t = lax.rem(me + delta, N_DEV)
            rdma = pltpu.make_async_remote_copy(
                src_ref=y_ref.at[t],
                dst_ref=back_ref.at[me],
                send_sem=back_send.at[t],
                recv_sem=back_recv.at[me],
                device_id=(t,),
                device_id_type=pl.DeviceIdType.MESH,
            )
            rdma.start()
            sends.append(rdma)
        for delta in range(1, N_DEV):
            s = lax.rem(me + N_DEV - delta, N_DEV)
            recv = pltpu.make_async_remote_copy(
                src_ref=y_ref.at[s],
                dst_ref=back_ref.at[s],
                send_sem=back_send.at[s],
                recv_sem=back_recv.at[s],
                device_id=(s,),
                device_id_type=pl.DeviceIdType.MESH,
            )
            recv.wait_recv()

        acc = shared
        for t in range(N_DEV):
            iota_t = lax.broadcasted_iota(jnp.int32, (n_tok, BLK), 1) + t * BLK
            d_t = (iota_t == slot_c).astype(jnp.bfloat16)
            yb_t = back_ref[t].reshape(BLK, h_dim)
            acc = acc + p_col * jnp.dot(
                d_t, yb_t, preferred_element_type=jnp.float32
            )
        out_ref[...] = acc

        for rdma in sends:
            rdma.wait_send()

    return pl.pallas_call(
        body,
        out_shape=jax.ShapeDtypeStruct((n_tok, h_dim), jnp.float32),
        in_specs=[pl.BlockSpec(memory_space=pltpu.VMEM)] * 6,
        out_specs=pl.BlockSpec(memory_space=pltpu.VMEM),
        scratch_shapes=[
            pltpu.VMEM((N_DEV, E_LOC, CAP, d), jnp.bfloat16),
            pltpu.VMEM((N_DEV, E_LOC, CAP, d), jnp.bfloat16),
            pltpu.VMEM((N_DEV, E_LOC, CAP, h_dim), jnp.bfloat16),
            pltpu.VMEM((N_DEV, E_LOC, CAP, h_dim), jnp.bfloat16),
            pltpu.SemaphoreType.DMA((N_DEV,)),
            pltpu.SemaphoreType.DMA((N_DEV,)),
            pltpu.SemaphoreType.DMA((N_DEV,)),
            pltpu.SemaphoreType.DMA((N_DEV,)),
        ],
        compiler_params=pltpu.CompilerParams(
            collective_id=0,
            vmem_limit_bytes=60 * 1024 * 1024,
        ),
    )(x, e_col, e_row, router_W, w_shard, shared_W)


def kernel(x, router_W, route_idx, expert_W, shared_W):
    e = route_idx.astype(jnp.int32)
    return _moe_fused(
        x, e, e.reshape(1, -1), router_W, expert_W, shared_W
    )


# device time: 67309 ns/iter; 2.4150x vs baseline; 1.1828x over previous
import jax
import jax.numpy as jnp
from jax import lax
from jax.experimental import pallas as pl
from jax.experimental.pallas import tpu as pltpu

N_DEV = 8
E_LOC = 4
CAP = 64
BLK = E_LOC * CAP


def _moe_fused(x, e_col, e_row, router_W, w_shard, shared_W):
    n_tok, d = x.shape
    e_loc, _, h_dim = w_shard.shape

    def body(x_ref, ec_ref, er_ref, rw_ref, w_ref, sw_ref, out_ref,
             bins_ref, r_ref, y_ref, back_ref,
             send_sems, recv_sems, back_send, back_recv):
        me = lax.axis_index("i")

        barrier_sem = pltpu.get_barrier_semaphore()
        for delta in range(1, N_DEV):
            pl.semaphore_signal(
                barrier_sem, inc=1,
                device_id=(lax.rem(me + delta, N_DEV),),
                device_id_type=pl.DeviceIdType.MESH,
            )

        e_c = ec_ref[...]
        e_r = er_ref[...]

        eq = (e_c == e_r).astype(jnp.float32)
        ii = lax.broadcasted_iota(jnp.int32, (n_tok, n_tok), 0)
        jj = lax.broadcasted_iota(jnp.int32, (n_tok, n_tok), 1)
        pos_c = (
            jnp.sum(eq * (jj <= ii).astype(jnp.float32), axis=1, keepdims=True)
            .astype(jnp.int32)
            - 1
        )
        pos_r = (
            jnp.sum(eq * (ii <= jj).astype(jnp.float32), axis=0, keepdims=True)
            .astype(jnp.int32)
            - 1
        )
        slot_c = e_c * CAP + jnp.minimum(pos_c, CAP - 1)
        slot_r = e_r * CAP + jnp.minimum(pos_r, CAP - 1)

        x_v = x_ref[...]
        for t in range(N_DEV):
            iota_t = lax.broadcasted_iota(jnp.int32, (BLK, n_tok), 0) + t * BLK
            d_t = (iota_t == slot_r).astype(jnp.float32)
            bins_ref[t] = (
                jnp.dot(d_t, x_v, preferred_element_type=jnp.float32)
                .astype(jnp.bfloat16)
                .reshape(E_LOC, CAP, d)
            )

        pl.semaphore_wait(barrier_sem, N_DEV - 1)

        r_ref[pl.ds(me, 1)] = bins_ref[pl.ds(me, 1)]
        sends = []
        for delta in range(1, N_DEV):
            t = lax.rem(me + delta, N_DEV)
            rdma = pltpu.make_async_remote_copy(
                src_ref=bins_ref.at[t],
                dst_ref=r_ref.at[me],
                send_sem=send_sems.at[t],
                recv_sem=recv_sems.at[me],
                device_id=(t,),
                device_id_type=pl.DeviceIdType.MESH,
            )
            rdma.start()
            sends.append(rdma)

        shared = jnp.dot(x_v, sw_ref[...], preferred_element_type=jnp.float32)
        scores = jnp.dot(x_v, rw_ref[...], preferred_element_type=jnp.float32)
        m = jnp.max(scores, axis=1, keepdims=True)
        ex = jnp.exp(scores - m)
        onehot_e = (
            e_c == lax.broadcasted_iota(jnp.int32, (n_tok, scores.shape[1]), 1)
        ).astype(jnp.float32)
        p_col = jnp.sum(ex * onehot_e, axis=1, keepdims=True) / jnp.sum(
            ex, axis=1, keepdims=True
        )

        for delta in range(1, N_DEV):
            s = lax.rem(me + N_DEV - delta, N_DEV)
            recv = pltpu.make_async_remote_copy(
                src_ref=bins_ref.at[s],
                dst_ref=r_ref.at[s],
                send_sem=send_sems.at[s],
                recv_sem=recv_sems.at[s],
                device_id=(s,),
                device_id_type=pl.DeviceIdType.MESH,
            )
            recv.wait_recv()

        for e in range(e_loc):
            x_e = r_ref[:, e].reshape(N_DEV * CAP, d).astype(jnp.float32)
            y_e = jnp.dot(x_e, w_ref[e], preferred_element_type=jnp.float32)
            y_ref[:, e] = y_e.astype(jnp.bfloat16).reshape(N_DEV, CAP, h_dim)
            back_ref[pl.ds(me, 1), e] = y_ref[pl.ds(me, 1), e]
            for delta in range(1, N_DEV):
                t = lax.rem(me + delta, N_DEV)
                rdma = pltpu.make_async_remote_copy(
                    src_ref=y_ref.at[t, e],
                    dst_ref=back_ref.at[me, e],
                    send_sem=back_send.at[t * E_LOC + e],
                    recv_sem=back_recv.at[me * E_LOC + e],
                    device_id=(t,),
                    device_id_type=pl.DeviceIdType.MESH,
                )
                rdma.start()
                sends.append(rdma)

        acc = shared
        for delta in range(N_DEV):
            s = me if delta == 0 else lax.rem(me + N_DEV - delta, N_DEV)
            if delta:
                for e in range(e_loc):
                    recv = pltpu.make_async_remote_copy(
                        src_ref=y_ref.at[s, e],
                        dst_ref=back_ref.at[s, e],
                        send_sem=back_send.at[s * E_LOC + e],
                        recv_sem=back_recv.at[s * E_LOC + e],
                        device_id=(s,),
                        device_id_type=pl.DeviceIdType.MESH,
                    )
                    recv.wait_recv()
            iota_s = (
                lax.broadcasted_iota(jnp.int32, (n_tok, BLK), 1) + s * BLK
            )
            d_s = (iota_s == slot_c).astype(jnp.bfloat16)
            yb_s = back_ref[pl.ds(s, 1)].reshape(BLK, h_dim)
            acc = acc + p_col * jnp.dot(
                d_s, yb_s, preferred_element_type=jnp.float32
            )
        out_ref[...] = acc

        for rdma in sends:
            rdma.wait_send()

    return pl.pallas_call(
        body,
        out_shape=jax.ShapeDtypeStruct((n_tok, h_dim), jnp.float32),
        in_specs=[pl.BlockSpec(memory_space=pltpu.VMEM)] * 6,
        out_specs=pl.BlockSpec(memory_space=pltpu.VMEM),
        scratch_shapes=[
            pltpu.VMEM((N_DEV, E_LOC, CAP, d), jnp.bfloat16),
            pltpu.VMEM((N_DEV, E_LOC, CAP, d), jnp.bfloat16),
            pltpu.VMEM((N_DEV, E_LOC, CAP, h_dim), jnp.bfloat16),
            pltpu.VMEM((N_DEV, E_LOC, CAP, h_dim), jnp.bfloat16),
            pltpu.SemaphoreType.DMA((N_DEV,)),
            pltpu.SemaphoreType.DMA((N_DEV,)),
            pltpu.SemaphoreType.DMA((N_DEV * E_LOC,)),
            pltpu.SemaphoreType.DMA((N_DEV * E_LOC,)),
        ],
        compiler_params=pltpu.CompilerParams(
            collective_id=0,
            vmem_limit_bytes=60 * 1024 * 1024,
        ),
    )(x, e_col, e_row, router_W, w_shard, shared_W)


def kernel(x, router_W, route_idx, expert_W, shared_W):
    e = route_idx.astype(jnp.int32)
    return _moe_fused(
        x, e, e.reshape(1, -1), router_W, expert_W, shared_W
    )


# device time: 67000 ns/iter; 2.4262x vs baseline; 1.0046x over previous
import jax
import jax.numpy as jnp
from jax import lax
from jax.experimental import pallas as pl
from jax.experimental.pallas import tpu as pltpu

N_DEV = 8
E_LOC = 4
CAP = 64
BLK = E_LOC * CAP


def _moe_fused(x, e_col, e_row, router_W, w_shard, shared_W):
    n_tok, d = x.shape
    e_loc, _, h_dim = w_shard.shape

    def body(x_ref, ec_ref, er_ref, rw_ref, w_ref, sw_ref, out_ref,
             bins_ref, r_ref, y_ref, back_ref,
             send_sems, recv_sems, back_send, back_recv):
        me = lax.axis_index("i")

        barrier_sem = pltpu.get_barrier_semaphore()
        for delta in range(1, N_DEV):
            pl.semaphore_signal(
                barrier_sem, inc=1,
                device_id=(lax.rem(me + delta, N_DEV),),
                device_id_type=pl.DeviceIdType.MESH,
            )

        e_c = ec_ref[...]
        e_r = er_ref[...]

        eq = (e_c == e_r).astype(jnp.float32)
        ii = lax.broadcasted_iota(jnp.int32, (n_tok, n_tok), 0)
        jj = lax.broadcasted_iota(jnp.int32, (n_tok, n_tok), 1)
        pos_c = (
            jnp.sum(eq * (jj <= ii).astype(jnp.float32), axis=1, keepdims=True)
            .astype(jnp.int32)
            - 1
        )
        pos_r = (
            jnp.sum(eq * (ii <= jj).astype(jnp.float32), axis=0, keepdims=True)
            .astype(jnp.int32)
            - 1
        )
        slot_c = e_c * CAP + jnp.minimum(pos_c, CAP - 1)
        slot_r = e_r * CAP + jnp.minimum(pos_r, CAP - 1)

        pl.semaphore_wait(barrier_sem, N_DEV - 1)

        x_v = x_ref[...]
        sends = []

        def build_bins(t_static, t_idx):
            iota_t = (
                lax.broadcasted_iota(jnp.int32, (BLK, n_tok), 0) + t_idx * BLK
            )
            d_t = (iota_t == slot_r).astype(jnp.float32)
            bins_ref[pl.ds(t_idx, 1)] = (
                jnp.dot(d_t, x_v, preferred_element_type=jnp.float32)
                .astype(jnp.bfloat16)
                .reshape(1, E_LOC, CAP, d)
            )

        for delta in range(1, N_DEV):
            t = lax.rem(me + delta, N_DEV)
            build_bins(delta, t)
            rdma = pltpu.make_async_remote_copy(
                src_ref=bins_ref.at[t],
                dst_ref=r_ref.at[me],
                send_sem=send_sems.at[t],
                recv_sem=recv_sems.at[me],
                device_id=(t,),
                device_id_type=pl.DeviceIdType.MESH,
            )
            rdma.start()
            sends.append(rdma)
        build_bins(0, me)
        r_ref[pl.ds(me, 1)] = bins_ref[pl.ds(me, 1)]

        shared = jnp.dot(x_v, sw_ref[...], preferred_element_type=jnp.float32)
        scores = jnp.dot(x_v, rw_ref[...], preferred_element_type=jnp.float32)
        m = jnp.max(scores, axis=1, keepdims=True)
        ex = jnp.exp(scores - m)
        onehot_e = (
            e_c == lax.broadcasted_iota(jnp.int32, (n_tok, scores.shape[1]), 1)
        ).astype(jnp.float32)
        p_col = jnp.sum(ex * onehot_e, axis=1, keepdims=True) / jnp.sum(
            ex, axis=1, keepdims=True
        )

        def process_source(s_idx, is_self):
            xs = r_ref[pl.ds(s_idx, 1)].reshape(E_LOC, CAP, d)
            ys = [
                jnp.dot(
                    xs[e].astype(jnp.float32), w_ref[e],
                    preferred_element_type=jnp.float32,
                ).astype(jnp.bfloat16)
                for e in range(e_loc)
            ]
            y_block = jnp.stack(ys, axis=0).reshape(1, E_LOC, CAP, h_dim)
            if is_self:
                back_ref[pl.ds(s_idx, 1)] = y_block
            else:
                y_ref[pl.ds(s_idx, 1)] = y_block

        process_source(me, True)
        for delta in range(1, N_DEV):
            s = lax.rem(me + N_DEV - delta, N_DEV)
            recv = pltpu.make_async_remote_copy(
                src_ref=bins_ref.at[s],
                dst_ref=r_ref.at[s],
                send_sem=send_sems.at[s],
                recv_sem=recv_sems.at[s],
                device_id=(s,),
                device_id_type=pl.DeviceIdType.MESH,
            )
            recv.wait_recv()
            process_source(s, False)
            rdma = pltpu.make_async_remote_copy(
                src_ref=y_ref.at[s],
                dst_ref=back_ref.at[me],
                send_sem=back_send.at[s],
                recv_sem=back_recv.at[me],
                device_id=(s,),
                device_id_type=pl.DeviceIdType.MESH,
            )
            rdma.start()
            sends.append(rdma)

        acc = shared

        def combine(s_idx):
            iota_s = (
                lax.broadcasted_iota(jnp.int32, (n_tok, BLK), 1) + s_idx * BLK
            )
            d_s = (iota_s == slot_c).astype(jnp.bfloat16)
            yb_s = back_ref[pl.ds(s_idx, 1)].reshape(BLK, h_dim)
            return acc + p_col * jnp.dot(
                d_s, yb_s, preferred_element_type=jnp.float32
            )

        acc = combine(me)
        for delta in range(1, N_DEV):
            s = lax.rem(me + delta, N_DEV)
            recv = pltpu.make_async_remote_copy(
                src_ref=y_ref.at[s],
                dst_ref=back_ref.at[s],
                send_sem=back_send.at[s],
                recv_sem=back_recv.at[s],
                device_id=(s,),
                device_id_type=pl.DeviceIdType.MESH,
            )
            recv.wait_recv()
            acc = combine(s)
        out_ref[...] = acc

        for rdma in sends:
            rdma.wait_send()

    return pl.pallas_call(
        body,
        out_shape=jax.ShapeDtypeStruct((n_tok, h_dim), jnp.float32),
        in_specs=[pl.BlockSpec(memory_space=pltpu.VMEM)] * 6,
        out_specs=pl.BlockSpec(memory_space=pltpu.VMEM),
        scratch_shapes=[
            pltpu.VMEM((N_DEV, E_LOC, CAP, d), jnp.bfloat16),
            pltpu.VMEM((N_DEV, E_LOC, CAP, d), jnp.bfloat16),
            pltpu.VMEM((N_DEV, E_LOC, CAP, h_dim), jnp.bfloat16),
            pltpu.VMEM((N_DEV, E_LOC, CAP, h_dim), jnp.bfloat16),
            pltpu.SemaphoreType.DMA((N_DEV,)),
            pltpu.SemaphoreType.DMA((N_DEV,)),
            pltpu.SemaphoreType.DMA((N_DEV,)),
            pltpu.SemaphoreType.DMA((N_DEV,)),
        ],
        compiler_params=pltpu.CompilerParams(
            collective_id=0,
            vmem_limit_bytes=60 * 1024 * 1024,
        ),
    )(x, e_col, e_row, router_W, w_shard, shared_W)


def kernel(x, router_W, route_idx, expert_W, shared_W):
    e = route_idx.astype(jnp.int32)
    return _moe_fused(
        x, e, e.reshape(1, -1), router_W, expert_W, shared_W
    )
